# baseline (speedup 1.0000x reference)
import functools

import jax
import jax.numpy as jnp
from jax import lax
from jax.experimental import pallas as pl
from jax.experimental.pallas import tpu as pltpu
from jax.experimental.pallas import tpu_sc as plsc

B = 4
N = 4096
S = 1024
K = 16
CIN = 128
COUT = 256
EPS = 1e-5

NROW = 32
NLANE = 128
NC = 2
NS = 16
NW = NC * NS
QPW = (B * S) // NW
SB = 256
RB = 2048
CB = 2048
FB = 1024


def _fps_body(xyz_ref, px_ref, py_ref, pz_ref):
    x = xyz_ref[:, 0]
    y = xyz_ref[:, 1]
    z = xyz_ref[:, 2]
    lx = x[:, 0:1, 0:1]
    ly = y[:, 0:1, 0:1]
    lz = z[:, 0:1, 0:1]
    px_ref[:, 0:1, :] = lx
    py_ref[:, 0:1, :] = ly
    pz_ref[:, 0:1, :] = lz
    dists0 = jnp.full((B, NROW, NLANE), jnp.inf, jnp.float32)

    def body(i, carry):
        dists, cx, cy, cz = carry
        xb = xyz_ref[:, 0]
        yb = xyz_ref[:, 1]
        zb = xyz_ref[:, 2]
        d = (xb - cx) ** 2 + (yb - cy) ** 2 + (zb - cz) ** 2
        dists = jnp.minimum(dists, d)
        m = jnp.max(dists, axis=(1, 2), keepdims=True)
        mask = dists == m
        nx0 = jnp.max(jnp.where(mask, xb, -jnp.inf), axis=(1, 2), keepdims=True)
        ny0 = jnp.max(jnp.where(mask, yb, -jnp.inf), axis=(1, 2), keepdims=True)
        nz0 = jnp.max(jnp.where(mask, zb, -jnp.inf), axis=(1, 2), keepdims=True)
        cnt = jnp.sum(mask.astype(jnp.int32), axis=(1, 2), keepdims=True)
        anytie = jnp.max(cnt) > 1

        def exact(_):
            li = (lax.broadcasted_iota(jnp.int32, (B, NROW, NLANE), 1) * NLANE
                  + lax.broadcasted_iota(jnp.int32, (B, NROW, NLANE), 2))
            nxt = jnp.min(jnp.where(mask, li, N), axis=(1, 2), keepdims=True)
            sel = li == nxt
            ex = jnp.max(jnp.where(sel, xb, -jnp.inf), axis=(1, 2),
                         keepdims=True)
            ey = jnp.max(jnp.where(sel, yb, -jnp.inf), axis=(1, 2),
                         keepdims=True)
            ez = jnp.max(jnp.where(sel, zb, -jnp.inf), axis=(1, 2),
                         keepdims=True)
            return ex, ey, ez

        def fast(_):
            return nx0, ny0, nz0

        nx, ny, nz = lax.cond(anytie, exact, fast, 0)
        px_ref[:, pl.ds(i, 1), :] = nx
        py_ref[:, pl.ds(i, 1), :] = ny
        pz_ref[:, pl.ds(i, 1), :] = nz
        return dists, nx, ny, nz

    lax.fori_loop(1, S, body, (dists0, lx, ly, lz))


def _fps(xyz_t):
    return pl.pallas_call(
        _fps_body,
        in_specs=[pl.BlockSpec((B, 3, NROW, NLANE), lambda: (0, 0, 0, 0))],
        out_specs=[pl.BlockSpec((B, S, 1), lambda: (0, 0, 0))] * 3,
        out_shape=[jax.ShapeDtypeStruct((B, S, 1), jnp.float32)] * 3,
    )(xyz_t)


GR = 128
NCHUNK = N // NLANE


def _knn_body(p1_ref, p2_ref, oidx_ref, dscr_ref):
    bid = pl.program_id(0)
    p1 = p1_ref[0]
    p2t = p2_ref[0]
    cross = jnp.dot(p1, p2t, preferred_element_type=jnp.float32)
    p2sq = jnp.sum(p2t * p2t, axis=0, keepdims=True)
    dscr_ref[...] = p2sq - 2.0 * cross
    li = lax.broadcasted_iota(jnp.int32, (GR, NLANE), 1)
    kcols = lax.broadcasted_iota(jnp.int32, (GR, K), 1)
    BIGI = jnp.int32(N)

    def group(g, _):
        rows = pl.ds(g * GR, GR)
        ids0 = jnp.zeros((GR, K), jnp.int32)
        gprev0 = jnp.full((GR, 1), -1, jnp.int32)

        def pick(k, carry):
            gprev, ids = carry
            acc = jnp.full((GR, NLANE), jnp.inf, jnp.float32)
            pay = jnp.zeros((GR, NLANE), jnp.int32)
            for c in range(NCHUNK):
                sl = pl.ds(c * NLANE, NLANE)
                lic = li + c * NLANE
                dd_c = dscr_ref[rows, sl]
                dd_c = jnp.where(lic == gprev, jnp.inf, dd_c)
                dscr_ref[rows, sl] = dd_c
                better = dd_c < acc
                pay = jnp.where(better, lic, pay)
                acc = jnp.minimum(acc, dd_c)
            m = jnp.min(acc, axis=1, keepdims=True)
            gidx = jnp.min(jnp.where(acc == m, pay, BIGI),
                           axis=1, keepdims=True)
            ids = jnp.where(kcols == k, jnp.broadcast_to(gidx, (GR, K)), ids)
            return gidx, ids

        _, ids = lax.fori_loop(0, K, pick, (gprev0, ids0))
        oidx_ref[0, rows, :] = ids + bid * N
        return 0

    lax.fori_loop(0, SB // GR, group, 0)


def _knn(p1p, p2tp):
    return pl.pallas_call(
        _knn_body,
        grid=(B, S // SB),
        in_specs=[
            pl.BlockSpec((1, SB, 8), lambda bb, sb: (bb, sb, 0)),
            pl.BlockSpec((1, 8, N), lambda bb, sb: (bb, 0, 0)),
        ],
        out_specs=pl.BlockSpec((1, SB, K), lambda bb, sb: (bb, sb, 0)),
        out_shape=jax.ShapeDtypeStruct((B, S, K), jnp.int32),
        scratch_shapes=[pltpu.VMEM((SB, N), jnp.float32)],
    )(p1p, p2tp)


def _proj_body(f_ref, wt_ref, b_ref, z_ref):
    z_ref[...] = (jnp.dot(f_ref[...], wt_ref[...],
                          preferred_element_type=jnp.float32) + b_ref[...])


def _proj(zf, wt, brow):
    return pl.pallas_call(
        _proj_body,
        grid=((B * N) // RB,),
        in_specs=[
            pl.BlockSpec((RB, CIN), lambda r: (r, 0)),
            pl.BlockSpec((CIN, COUT), lambda r: (0, 0)),
            pl.BlockSpec((1, COUT), lambda r: (0, 0)),
        ],
        out_specs=pl.BlockSpec((RB, COUT), lambda r: (r, 0)),
        out_shape=jax.ShapeDtypeStruct((B * N, COUT), jnp.float32),
    )(zf, wt, brow)


def _gather_body(z_hbm, idx_hbm, m_hbm, hist_hbm,
                 idxall_v, rows_v, mbuf_v, hist_v, sem0, sem1):
    c = lax.axis_index("c")
    s = lax.axis_index("s")
    wid = s * NC + c
    base = pl.multiple_of(wid * QPW, QPW)

    zf = jnp.zeros((16,), jnp.float32)

    def zb(i, _):
        hist_v[pl.ds(pl.multiple_of(i * 16, 16), 16)] = zf
        return 0

    lax.fori_loop(0, (B * N) // 16, zb, 0)

    pltpu.sync_copy(idx_hbm.at[pl.ds(base, QPW)], idxall_v)
    ones = jnp.ones((16,), jnp.float32)

    pltpu.async_copy(z_hbm.at[idxall_v.at[0]], rows_v.at[0], sem0)
    pltpu.async_copy(z_hbm.at[idxall_v.at[1]], rows_v.at[1], sem1)

    def qb(p, _):
        for s_i, sem_i in ((0, sem0), (1, sem1)):
            q = p * 2 + s_i
            pltpu.make_async_copy(
                z_hbm.at[idxall_v.at[q]], rows_v.at[s_i], sem_i).wait()
            for cc in range(COUT // 16):
                sl = pl.ds(cc * 16, 16)
                acc = jnp.maximum(rows_v[s_i, 0, sl], rows_v[s_i, 1, sl])
                for r in range(2, K):
                    acc = jnp.maximum(acc, rows_v[s_i, r, sl])
                mbuf_v[q, sl] = acc
            iv = idxall_v[q]
            plsc.addupdate_scatter(hist_v, [iv], ones)
            nq = q + 2

            @pl.when(nq < QPW)
            def _():
                pltpu.async_copy(
                    z_hbm.at[idxall_v.at[nq]], rows_v.at[s_i], sem_i)

        return 0

    lax.fori_loop(0, QPW // 2, qb, 0)
    pltpu.sync_copy(mbuf_v, m_hbm.at[pl.ds(base, QPW)])
    pltpu.sync_copy(hist_v, hist_hbm.at[wid])


def _gather(zz, idx_flat):
    mesh = plsc.VectorSubcoreMesh(core_axis_name="c", subcore_axis_name="s",
                                  num_cores=NC, num_subcores=NS)
    f = pl.kernel(
        _gather_body,
        out_type=[jax.ShapeDtypeStruct((B * S, COUT), jnp.float32),
                  jax.ShapeDtypeStruct((NW, B * N), jnp.float32)],
        mesh=mesh,
        compiler_params=pltpu.CompilerParams(needs_layout_passes=False),
        scratch_types=[
            pltpu.VMEM((QPW, K), jnp.int32),
            pltpu.VMEM((2, K, COUT), jnp.float32),
            pltpu.VMEM((QPW, COUT), jnp.float32),
            pltpu.VMEM((B * N,), jnp.float32),
            pltpu.SemaphoreType.DMA,
            pltpu.SemaphoreType.DMA,
        ],
    )
    return f(zz, idx_flat)


def _stats_body(hist_ref, z_ref, g_ref, be_ref, scale_ref, shift_ref, acc_ref):
    kb = pl.program_id(0)

    @pl.when(kb == 0)
    def _():
        acc_ref[...] = jnp.zeros_like(acc_ref)

    cc = jnp.sum(hist_ref[...], axis=0, keepdims=True)
    zb = z_ref[...]
    acc_ref[0:1, :] += jnp.dot(cc, zb, preferred_element_type=jnp.float32)
    acc_ref[1:2, :] += jnp.dot(cc, zb * zb,
                               preferred_element_type=jnp.float32)

    @pl.when(kb == (B * N) // CB - 1)
    def _():
        tot = float(B * S * K)
        mean = acc_ref[0:1, :] * (1.0 / tot)
        ex2 = acc_ref[1:2, :] * (1.0 / tot)
        var = ex2 - mean * mean
        inv = lax.rsqrt(var + EPS)
        scale = g_ref[...] * inv
        shift = be_ref[...] - mean * scale
        scale_ref[...] = jnp.broadcast_to(scale, (8, COUT))
        shift_ref[...] = jnp.broadcast_to(shift, (8, COUT))


def _stats(hist, zz, grow, berow):
    return pl.pallas_call(
        _stats_body,
        grid=((B * N) // CB,),
        in_specs=[
            pl.BlockSpec((NW, CB), lambda r: (0, r)),
            pl.BlockSpec((CB, COUT), lambda r: (r, 0)),
            pl.BlockSpec((1, COUT), lambda r: (0, 0)),
            pl.BlockSpec((1, COUT), lambda r: (0, 0)),
        ],
        out_specs=[pl.BlockSpec((8, COUT), lambda r: (0, 0))] * 2,
        out_shape=[jax.ShapeDtypeStruct((8, COUT), jnp.float32)] * 2,
        scratch_shapes=[pltpu.VMEM((8, COUT), jnp.float32)],
    )(hist, zz, grow, berow)


def _fin_body(m_ref, scale_ref, shift_ref, o_ref):
    o_ref[...] = jnp.maximum(
        m_ref[...] * scale_ref[0:1, :] + shift_ref[0:1, :], 0.0)


def _fin(m, scale, shift):
    return pl.pallas_call(
        _fin_body,
        grid=((B * S) // FB,),
        in_specs=[
            pl.BlockSpec((FB, COUT), lambda r: (r, 0)),
            pl.BlockSpec((8, COUT), lambda r: (0, 0)),
            pl.BlockSpec((8, COUT), lambda r: (0, 0)),
        ],
        out_specs=pl.BlockSpec((FB, COUT), lambda r: (r, 0)),
        out_shape=jax.ShapeDtypeStruct((B * S, COUT), jnp.float32),
    )(m, scale, shift)


def kernel(points_xyz, points_features, W, b, gamma, beta):
    xyz_t = jnp.transpose(points_xyz, (0, 2, 1)).reshape(B, 3, NROW, NLANE)
    px, py, pz = _fps(xyz_t)
    sampled = jnp.concatenate([px, py, pz], axis=-1)

    p1p = jnp.concatenate(
        [sampled, jnp.zeros((B, S, 5), jnp.float32)], axis=-1)
    p2tp = jnp.concatenate(
        [xyz_t.reshape(B, 3, N), jnp.zeros((B, 5, N), jnp.float32)], axis=1)
    knn_idx = _knn(p1p, p2tp)
    idx_flat = knn_idx.reshape(B * S, K)

    zf = points_features.reshape(B * N, CIN)
    zz = _proj(zf, W.T, b.reshape(1, COUT))

    m, hist = _gather(zz, idx_flat)

    scale, shift = _stats(hist, zz, gamma.reshape(1, COUT),
                          beta.reshape(1, COUT))
    out = _fin(m, scale, shift).reshape(B, S, COUT)
    return (sampled, out)


# speedup vs baseline: 1.0097x; 1.0097x over previous
import functools

import jax
import jax.numpy as jnp
from jax import lax
from jax.experimental import pallas as pl
from jax.experimental.pallas import tpu as pltpu
from jax.experimental.pallas import tpu_sc as plsc

B = 4
N = 4096
S = 1024
K = 16
CIN = 128
COUT = 256
EPS = 1e-5

NROW = 32
NLANE = 128
NC = 2
NS = 16
NW = NC * NS
QPW = (B * S) // NW
SB = 256
RB = 2048
CB = 2048
FB = 1024


def _fps_body(xyz_ref, px_ref, py_ref, pz_ref):
    x = xyz_ref[:, 0]
    y = xyz_ref[:, 1]
    z = xyz_ref[:, 2]
    lx = x[:, 0:1, 0:1]
    ly = y[:, 0:1, 0:1]
    lz = z[:, 0:1, 0:1]
    px_ref[:, 0:1, :] = lx
    py_ref[:, 0:1, :] = ly
    pz_ref[:, 0:1, :] = lz
    dists0 = jnp.full((B, NROW, NLANE), jnp.inf, jnp.float32)

    def body(i, carry):
        dists, cx, cy, cz = carry
        xb = xyz_ref[:, 0]
        yb = xyz_ref[:, 1]
        zb = xyz_ref[:, 2]
        d = (xb - cx) ** 2 + (yb - cy) ** 2 + (zb - cz) ** 2
        dists = jnp.minimum(dists, d)
        m = jnp.max(dists, axis=(1, 2), keepdims=True)
        mask = dists == m
        nx0 = jnp.max(jnp.where(mask, xb, -jnp.inf), axis=(1, 2), keepdims=True)
        ny0 = jnp.max(jnp.where(mask, yb, -jnp.inf), axis=(1, 2), keepdims=True)
        nz0 = jnp.max(jnp.where(mask, zb, -jnp.inf), axis=(1, 2), keepdims=True)
        cnt = jnp.sum(mask.astype(jnp.int32), axis=(1, 2), keepdims=True)
        anytie = jnp.max(cnt) > 1

        def exact(_):
            li = (lax.broadcasted_iota(jnp.int32, (B, NROW, NLANE), 1) * NLANE
                  + lax.broadcasted_iota(jnp.int32, (B, NROW, NLANE), 2))
            nxt = jnp.min(jnp.where(mask, li, N), axis=(1, 2), keepdims=True)
            sel = li == nxt
            ex = jnp.max(jnp.where(sel, xb, -jnp.inf), axis=(1, 2),
                         keepdims=True)
            ey = jnp.max(jnp.where(sel, yb, -jnp.inf), axis=(1, 2),
                         keepdims=True)
            ez = jnp.max(jnp.where(sel, zb, -jnp.inf), axis=(1, 2),
                         keepdims=True)
            return ex, ey, ez

        def fast(_):
            return nx0, ny0, nz0

        nx, ny, nz = lax.cond(anytie, exact, fast, 0)
        px_ref[:, pl.ds(i, 1), :] = nx
        py_ref[:, pl.ds(i, 1), :] = ny
        pz_ref[:, pl.ds(i, 1), :] = nz
        return dists, nx, ny, nz

    lax.fori_loop(1, S, body, (dists0, lx, ly, lz))


def _fps(xyz_t):
    return pl.pallas_call(
        _fps_body,
        in_specs=[pl.BlockSpec((B, 3, NROW, NLANE), lambda: (0, 0, 0, 0))],
        out_specs=[pl.BlockSpec((B, S, 1), lambda: (0, 0, 0))] * 3,
        out_shape=[jax.ShapeDtypeStruct((B, S, 1), jnp.float32)] * 3,
    )(xyz_t)


GR = 128
NCHUNK = N // NLANE


def _knn_body(p1_ref, p2_ref, oidx_ref, dscr_ref):
    bid = pl.program_id(0)
    p1 = p1_ref[0]
    p2t = p2_ref[0]
    cross = jnp.dot(p1, p2t, preferred_element_type=jnp.float32)
    p2sq = jnp.sum(p2t * p2t, axis=0, keepdims=True)
    dscr_ref[...] = p2sq - 2.0 * cross
    li = lax.broadcasted_iota(jnp.int32, (GR, NLANE), 1)
    kcols = lax.broadcasted_iota(jnp.int32, (GR, K), 1)
    BIGI = jnp.int32(N)

    def group(g, _):
        rows = pl.ds(g * GR, GR)
        ids0 = jnp.zeros((GR, K), jnp.int32)
        gprev0 = jnp.full((GR, 1), -1, jnp.int32)

        def pick(k, carry):
            gprev, ids = carry
            acc = jnp.full((GR, NLANE), jnp.inf, jnp.float32)
            pay = jnp.zeros((GR, NLANE), jnp.int32)
            for c in range(NCHUNK):
                sl = pl.ds(c * NLANE, NLANE)
                lic = li + c * NLANE
                dd_c = dscr_ref[rows, sl]
                dd_c = jnp.where(lic == gprev, jnp.inf, dd_c)
                dscr_ref[rows, sl] = dd_c
                better = dd_c < acc
                pay = jnp.where(better, lic, pay)
                acc = jnp.minimum(acc, dd_c)
            m = jnp.min(acc, axis=1, keepdims=True)
            gidx = jnp.min(jnp.where(acc == m, pay, BIGI),
                           axis=1, keepdims=True)
            ids = jnp.where(kcols == k, jnp.broadcast_to(gidx, (GR, K)), ids)
            return gidx, ids

        _, ids = lax.fori_loop(0, K, pick, (gprev0, ids0))
        oidx_ref[0, rows, :] = ids + bid * N
        return 0

    lax.fori_loop(0, SB // GR, group, 0)


def _knn(p1p, p2tp):
    return pl.pallas_call(
        _knn_body,
        grid=(B, S // SB),
        in_specs=[
            pl.BlockSpec((1, SB, 8), lambda bb, sb: (bb, sb, 0)),
            pl.BlockSpec((1, 8, N), lambda bb, sb: (bb, 0, 0)),
        ],
        out_specs=pl.BlockSpec((1, SB, K), lambda bb, sb: (bb, sb, 0)),
        out_shape=jax.ShapeDtypeStruct((B, S, K), jnp.int32),
        scratch_shapes=[pltpu.VMEM((SB, N), jnp.float32)],
    )(p1p, p2tp)


def _proj_body(f_ref, wt_ref, b_ref, z_ref):
    z_ref[...] = (jnp.dot(f_ref[...], wt_ref[...],
                          preferred_element_type=jnp.float32) + b_ref[...])


def _proj(zf, wt, brow):
    return pl.pallas_call(
        _proj_body,
        grid=((B * N) // RB,),
        in_specs=[
            pl.BlockSpec((RB, CIN), lambda r: (r, 0)),
            pl.BlockSpec((CIN, COUT), lambda r: (0, 0)),
            pl.BlockSpec((1, COUT), lambda r: (0, 0)),
        ],
        out_specs=pl.BlockSpec((RB, COUT), lambda r: (r, 0)),
        out_shape=jax.ShapeDtypeStruct((B * N, COUT), jnp.float32),
    )(zf, wt, brow)


def _gather_body(z_hbm, idx_hbm, m_hbm, sums_hbm,
                 idxall_v, rows_v, mbuf_v, sacc_v, sem0, sem1):
    c = lax.axis_index("c")
    s = lax.axis_index("s")
    wid = s * NC + c
    base = pl.multiple_of(wid * QPW, QPW)

    zf = jnp.zeros((16,), jnp.float32)
    for cc in range(COUT // 16):
        sacc_v[0, pl.ds(cc * 16, 16)] = zf
        sacc_v[1, pl.ds(cc * 16, 16)] = zf

    pltpu.sync_copy(idx_hbm.at[pl.ds(base, QPW)], idxall_v)

    pltpu.async_copy(z_hbm.at[idxall_v.at[0]], rows_v.at[0], sem0)
    pltpu.async_copy(z_hbm.at[idxall_v.at[1]], rows_v.at[1], sem1)

    def qb(p, _):
        for s_i, sem_i in ((0, sem0), (1, sem1)):
            q = p * 2 + s_i
            pltpu.make_async_copy(
                z_hbm.at[idxall_v.at[q]], rows_v.at[s_i], sem_i).wait()
            for cc in range(COUT // 16):
                sl = pl.ds(cc * 16, 16)
                r0 = rows_v[s_i, 0, sl]
                r1 = rows_v[s_i, 1, sl]
                mx = jnp.maximum(r0, r1)
                sm = r0 + r1
                sq = r0 * r0 + r1 * r1
                for r in range(2, K):
                    rr = rows_v[s_i, r, sl]
                    mx = jnp.maximum(mx, rr)
                    sm = sm + rr
                    sq = sq + rr * rr
                mbuf_v[q, sl] = mx
                plsc.addupdate(sacc_v.at[0, sl], sm)
                plsc.addupdate(sacc_v.at[1, sl], sq)
            nq = q + 2

            @pl.when(nq < QPW)
            def _():
                pltpu.async_copy(
                    z_hbm.at[idxall_v.at[nq]], rows_v.at[s_i], sem_i)

        return 0

    lax.fori_loop(0, QPW // 2, qb, 0)
    pltpu.sync_copy(mbuf_v, m_hbm.at[pl.ds(base, QPW)])
    pltpu.sync_copy(sacc_v, sums_hbm.at[wid])


def _gather(zz, idx_flat):
    mesh = plsc.VectorSubcoreMesh(core_axis_name="c", subcore_axis_name="s",
                                  num_cores=NC, num_subcores=NS)
    f = pl.kernel(
        _gather_body,
        out_type=[jax.ShapeDtypeStruct((B * S, COUT), jnp.float32),
                  jax.ShapeDtypeStruct((NW, 2, COUT), jnp.float32)],
        mesh=mesh,
        compiler_params=pltpu.CompilerParams(needs_layout_passes=False),
        scratch_types=[
            pltpu.VMEM((QPW, K), jnp.int32),
            pltpu.VMEM((2, K, COUT), jnp.float32),
            pltpu.VMEM((QPW, COUT), jnp.float32),
            pltpu.VMEM((2, COUT), jnp.float32),
            pltpu.SemaphoreType.DMA,
            pltpu.SemaphoreType.DMA,
        ],
    )
    return f(zz, idx_flat)


def _fin_body(sums_ref, m_ref, g_ref, be_ref, o_ref, aff_ref):
    kb = pl.program_id(0)

    @pl.when(kb == 0)
    def _():
        tot = float(B * S * K)
        mean = jnp.sum(sums_ref[:, 0, :], axis=0, keepdims=True) * (1.0 / tot)
        ex2 = jnp.sum(sums_ref[:, 1, :], axis=0, keepdims=True) * (1.0 / tot)
        var = ex2 - mean * mean
        inv = lax.rsqrt(var + EPS)
        scale = g_ref[...] * inv
        shift = be_ref[...] - mean * scale
        aff_ref[0:1, :] = scale
        aff_ref[1:2, :] = shift

    o_ref[...] = jnp.maximum(
        m_ref[...] * aff_ref[0:1, :] + aff_ref[1:2, :], 0.0)


def _fin(m, sums, grow, berow):
    return pl.pallas_call(
        _fin_body,
        grid=((B * S) // FB,),
        in_specs=[
            pl.BlockSpec((NW, 2, COUT), lambda r: (0, 0, 0)),
            pl.BlockSpec((FB, COUT), lambda r: (r, 0)),
            pl.BlockSpec((1, COUT), lambda r: (0, 0)),
            pl.BlockSpec((1, COUT), lambda r: (0, 0)),
        ],
        out_specs=pl.BlockSpec((FB, COUT), lambda r: (r, 0)),
        out_shape=jax.ShapeDtypeStruct((B * S, COUT), jnp.float32),
        scratch_shapes=[pltpu.VMEM((2, COUT), jnp.float32)],
    )(sums, m, grow, berow)


def kernel(points_xyz, points_features, W, b, gamma, beta):
    xyz_t = jnp.transpose(points_xyz, (0, 2, 1)).reshape(B, 3, NROW, NLANE)
    px, py, pz = _fps(xyz_t)
    sampled = jnp.concatenate([px, py, pz], axis=-1)

    p1p = jnp.concatenate(
        [sampled, jnp.zeros((B, S, 5), jnp.float32)], axis=-1)
    p2tp = jnp.concatenate(
        [xyz_t.reshape(B, 3, N), jnp.zeros((B, 5, N), jnp.float32)], axis=1)
    knn_idx = _knn(p1p, p2tp)
    idx_flat = knn_idx.reshape(B * S, K)

    zf = points_features.reshape(B * N, CIN)
    zz = _proj(zf, W.T, b.reshape(1, COUT))

    m, sums = _gather(zz, idx_flat)

    out = _fin(m, sums, gamma.reshape(1, COUT),
               beta.reshape(1, COUT)).reshape(B, S, COUT)
    return (sampled, out)
